# baseline (speedup 1.0000x reference)
import jax
import jax.numpy as jnp
from jax import lax
from jax.experimental import pallas as pl
from jax.experimental.pallas import tpu as pltpu

D = 128
K = 65536
BP = 16384
BS = 16384
C = 4096
NG = K // C
NBS = BS // C


def _win(g, ptr, B):
    t0 = (g * C - ptr) % K
    in_win = (t0 < B) | (t0 > K - C)
    full_in = t0 <= B - C
    o = jnp.where(t0 < B, t0, t0 - K)
    oc = jnp.clip(o, 0, B - C)
    return t0, in_win, full_in, o, oc


def _qmap_p(g, pp, ps):
    _, _, full_in, _, _ = _win(g, pp[0], BP)
    return (0, jnp.where(full_in, 0, g))


def _qmap_s(g, pp, ps):
    _, _, full_in, _, _ = _win(g, ps[0], BS)
    return (0, jnp.where(full_in, 0, g))


def _klmap(g, pp, ps):
    _, _, full_in, _, _ = _win(g, ps[0], BS)
    return (jnp.where(full_in, 0, g), 0, 0)


def _seg_maps():
    def base(g, pp, ps):
        t0, in_win, _, o, _ = _win(g, ps[0], BS)
        return jnp.where(in_win, o // C, 0)

    def amap(g, pp, ps):
        return (base(g, pp, ps) % NBS, 0, 0)

    def bmap(g, pp, ps):
        return ((base(g, pp, ps) + 1) % NBS, 0, 0)

    return amap, bmap


_sl_a, _sl_b = _seg_maps()


def _body(pp_ref, ps_ref, qp_ref, qs_ref, kp_hbm, ks_hbm, kl_ref, sl1, sl2,
          out_ref, lab_ref, kbuf_p, kbuf_s, sem_p, sem_s):
    g = pl.program_id(0)
    c0 = g * C
    col = lax.broadcasted_iota(jnp.int32, (1, C), 1) + c0
    ptr_p = pp_ref[0]
    ptr_s = ps_ref[0]

    def issue(gg, ptr, B, khbm, kbuf, sem):
        t0 = (gg * C - ptr) % K
        in_win = (t0 < B) | (t0 > K - C)

        @pl.when(in_win & (gg < NG))
        def _():
            o = jnp.where(t0 < B, t0, t0 - K)
            oc = jnp.clip(o, 0, B - C)
            pltpu.make_async_copy(
                khbm.at[pl.ds(oc, C), :], kbuf.at[gg % 2], sem.at[gg % 2]
            ).start()

    @pl.when(g == 0)
    def _():
        issue(0, ptr_p, BP, kp_hbm, kbuf_p, sem_p)
        issue(0, ptr_s, BS, ks_hbm, kbuf_s, sem_s)

    issue(g + 1, ptr_p, BP, kp_hbm, kbuf_p, sem_p)
    issue(g + 1, ptr_s, BS, ks_hbm, kbuf_s, sem_s)

    def enqueue_half(ptr, B, khbm, kbuf, sem, q_ref, row0):
        t0, in_win, full_in, o, oc = _win(g, ptr, B)

        def normalized(s):
            ssq = jnp.sum(s * s, axis=1, keepdims=True)
            return s * (1.0 / (jnp.sqrt(ssq) + 1e-12))

        def wait():
            pltpu.make_async_copy(
                khbm.at[pl.ds(oc, C), :], kbuf.at[g % 2], sem.at[g % 2]
            ).wait()

        @pl.when(full_in)
        def _():
            wait()
            out_ref[row0:row0 + D, :] = normalized(kbuf[g % 2]).T

        @pl.when(in_win & jnp.logical_not(full_in))
        def _():
            wait()
            resid = o - oc
            raw = kbuf[g % 2]
            s = lax.cond(
                resid == 0,
                lambda: raw,
                lambda: pltpu.roll(raw, -resid, 0),
            )
            sn = normalized(s)
            t = (col - ptr) % K
            out_ref[row0:row0 + D, :] = jnp.where(t < B, sn.T, q_ref[...])

        @pl.when(jnp.logical_not(in_win))
        def _():
            out_ref[row0:row0 + D, :] = q_ref[...]

        return in_win

    enqueue_half(ptr_p, BP, kp_hbm, kbuf_p, sem_p, qp_ref, 0)
    inws = enqueue_half(ptr_s, BS, ks_hbm, kbuf_s, sem_s, qs_ref, D)

    @pl.when(inws)
    def _():
        rs = (-ptr_s) % C
        ts = (col - ptr_s) % K
        lcat = jnp.concatenate([sl1[0], sl2[0]], axis=1)
        lsl = pltpu.roll(lcat, -rs, 1)[:, 0:C]
        lab_ref[0] = jnp.where(ts < BS, lsl, kl_ref[0])

    @pl.when(jnp.logical_not(inws))
    def _():
        lab_ref[0] = kl_ref[0]


def _make_call(interpret=False):
    grid_spec = pltpu.PrefetchScalarGridSpec(
        num_scalar_prefetch=2,
        grid=(NG,),
        in_specs=[
            pl.BlockSpec((D, C), _qmap_p),
            pl.BlockSpec((D, C), _qmap_s),
            pl.BlockSpec(memory_space=pl.MemorySpace.ANY),
            pl.BlockSpec(memory_space=pl.MemorySpace.ANY),
            pl.BlockSpec((1, 1, C), _klmap),
            pl.BlockSpec((1, 1, C), _sl_a),
            pl.BlockSpec((1, 1, C), _sl_b),
        ],
        out_specs=[
            pl.BlockSpec((2 * D, C), lambda g, pp, ps: (0, g)),
            pl.BlockSpec((1, 1, C), lambda g, pp, ps: (g, 0, 0)),
        ],
        scratch_shapes=[
            pltpu.VMEM((2, C, D), jnp.float32),
            pltpu.VMEM((2, C, D), jnp.float32),
            pltpu.SemaphoreType.DMA((2,)),
            pltpu.SemaphoreType.DMA((2,)),
        ],
    )
    return pl.pallas_call(
        _body,
        grid_spec=grid_spec,
        out_shape=[
            jax.ShapeDtypeStruct((2 * D, K), jnp.float32),
            jax.ShapeDtypeStruct((NG, 1, C), jnp.int32),
        ],
        interpret=interpret,
    )


def kernel(queue_pcd, queue_seg, keys_pcd, keys_seg, k_labels, seg_labels,
           ptr_pcd, ptr_seg):
    pp = jnp.asarray(ptr_pcd, jnp.int32).reshape(1)
    ps = jnp.asarray(ptr_seg, jnp.int32).reshape(1)
    kl3 = k_labels.reshape(NG, 1, C)
    seg3 = seg_labels.reshape(NBS, 1, C)
    queues, labels = _make_call()(
        pp, ps, queue_pcd, queue_seg, keys_pcd, keys_seg, kl3, seg3, seg3)
    new_ptr_pcd = ((jnp.asarray(ptr_pcd, jnp.int32) + BP) % K).astype(jnp.int32)
    new_ptr_seg = ((jnp.asarray(ptr_seg, jnp.int32) + BS) % K).astype(jnp.int32)
    return (queues, labels.reshape(K), new_ptr_pcd, new_ptr_seg)


# speedup vs baseline: 1.1059x; 1.1059x over previous
import jax
import jax.numpy as jnp
from jax import lax
from jax.experimental import pallas as pl
from jax.experimental.pallas import tpu as pltpu

D = 128
K = 65536
BP = 16384
BS = 16384
C = 2048
NG = K // C
NBS = BS // C


def _win(g, ptr, B):
    t0 = (g * C - ptr) % K
    in_win = (t0 < B) | (t0 > K - C)
    full_in = t0 <= B - C
    o = jnp.where(t0 < B, t0, t0 - K)
    oc = jnp.clip(o, 0, B - C)
    return t0, in_win, full_in, o, oc


def _qmap_p(g, pp, ps):
    _, _, full_in, _, _ = _win(g, pp[0], BP)
    return (0, jnp.where(full_in, 0, g))


def _qmap_s(g, pp, ps):
    _, _, full_in, _, _ = _win(g, ps[0], BS)
    return (0, jnp.where(full_in, 0, g))


def _klmap(g, pp, ps):
    _, _, full_in, _, _ = _win(g, ps[0], BS)
    return (jnp.where(full_in, 0, g), 0, 0)


def _seg_maps():
    def base(g, pp, ps):
        t0, in_win, _, o, _ = _win(g, ps[0], BS)
        return jnp.where(in_win, o // C, 0)

    def amap(g, pp, ps):
        return (base(g, pp, ps) % NBS, 0, 0)

    def bmap(g, pp, ps):
        return ((base(g, pp, ps) + 1) % NBS, 0, 0)

    return amap, bmap


_sl_a, _sl_b = _seg_maps()


def _body(pp_ref, ps_ref, qp_ref, qs_ref, kp_hbm, ks_hbm, kl_ref, sl1, sl2,
          out_ref, lab_ref, kbuf_p, kbuf_s, sem_p, sem_s):
    g = pl.program_id(0)
    c0 = g * C
    col = lax.broadcasted_iota(jnp.int32, (1, C), 1) + c0
    ptr_p = pp_ref[0]
    ptr_s = ps_ref[0]

    def issue(gg, ptr, B, khbm, kbuf, sem):
        t0 = (gg * C - ptr) % K
        in_win = (t0 < B) | (t0 > K - C)

        @pl.when(in_win & (gg < NG))
        def _():
            o = jnp.where(t0 < B, t0, t0 - K)
            oc = jnp.clip(o, 0, B - C)
            pltpu.make_async_copy(
                khbm.at[pl.ds(oc, C), :], kbuf.at[gg % 3], sem.at[gg % 3]
            ).start()

    @pl.when(g == 0)
    def _():
        issue(0, ptr_p, BP, kp_hbm, kbuf_p, sem_p)
        issue(0, ptr_s, BS, ks_hbm, kbuf_s, sem_s)
        issue(1, ptr_p, BP, kp_hbm, kbuf_p, sem_p)
        issue(1, ptr_s, BS, ks_hbm, kbuf_s, sem_s)

    issue(g + 2, ptr_p, BP, kp_hbm, kbuf_p, sem_p)
    issue(g + 2, ptr_s, BS, ks_hbm, kbuf_s, sem_s)

    def enqueue_half(ptr, B, khbm, kbuf, sem, q_ref, row0):
        t0, in_win, full_in, o, oc = _win(g, ptr, B)

        def normalized_t(s):
            t = s.T
            ssq = jnp.sum(t * t, axis=0, keepdims=True)
            return t * (1.0 / (jnp.sqrt(ssq) + 1e-12))

        def wait():
            pltpu.make_async_copy(
                khbm.at[pl.ds(oc, C), :], kbuf.at[g % 3], sem.at[g % 3]
            ).wait()

        @pl.when(full_in)
        def _():
            wait()
            out_ref[row0:row0 + D, :] = normalized_t(kbuf[g % 3])

        @pl.when(in_win & jnp.logical_not(full_in))
        def _():
            wait()
            resid = o - oc
            raw = kbuf[g % 3]
            s = lax.cond(
                resid == 0,
                lambda: raw,
                lambda: pltpu.roll(raw, -resid, 0),
            )
            sn = normalized_t(s)
            t = (col - ptr) % K
            out_ref[row0:row0 + D, :] = jnp.where(t < B, sn, q_ref[...])

        @pl.when(jnp.logical_not(in_win))
        def _():
            out_ref[row0:row0 + D, :] = q_ref[...]

        return in_win

    enqueue_half(ptr_p, BP, kp_hbm, kbuf_p, sem_p, qp_ref, 0)
    inws = enqueue_half(ptr_s, BS, ks_hbm, kbuf_s, sem_s, qs_ref, D)

    @pl.when(inws)
    def _():
        rs = (-ptr_s) % C
        ts = (col - ptr_s) % K
        lcat = jnp.concatenate([sl1[0], sl2[0]], axis=1)
        lsl = pltpu.roll(lcat, -rs, 1)[:, 0:C]
        lab_ref[0] = jnp.where(ts < BS, lsl, kl_ref[0])

    @pl.when(jnp.logical_not(inws))
    def _():
        lab_ref[0] = kl_ref[0]


def _make_call(interpret=False):
    grid_spec = pltpu.PrefetchScalarGridSpec(
        num_scalar_prefetch=2,
        grid=(NG,),
        in_specs=[
            pl.BlockSpec((D, C), _qmap_p),
            pl.BlockSpec((D, C), _qmap_s),
            pl.BlockSpec(memory_space=pl.MemorySpace.ANY),
            pl.BlockSpec(memory_space=pl.MemorySpace.ANY),
            pl.BlockSpec((1, 1, C), _klmap),
            pl.BlockSpec((1, 1, C), _sl_a),
            pl.BlockSpec((1, 1, C), _sl_b),
        ],
        out_specs=[
            pl.BlockSpec((2 * D, C), lambda g, pp, ps: (0, g)),
            pl.BlockSpec((1, 1, C), lambda g, pp, ps: (g, 0, 0)),
        ],
        scratch_shapes=[
            pltpu.VMEM((3, C, D), jnp.float32),
            pltpu.VMEM((3, C, D), jnp.float32),
            pltpu.SemaphoreType.DMA((3,)),
            pltpu.SemaphoreType.DMA((3,)),
        ],
    )
    return pl.pallas_call(
        _body,
        grid_spec=grid_spec,
        out_shape=[
            jax.ShapeDtypeStruct((2 * D, K), jnp.float32),
            jax.ShapeDtypeStruct((NG, 1, C), jnp.int32),
        ],
        interpret=interpret,
    )


def kernel(queue_pcd, queue_seg, keys_pcd, keys_seg, k_labels, seg_labels,
           ptr_pcd, ptr_seg):
    pp = jnp.asarray(ptr_pcd, jnp.int32).reshape(1)
    ps = jnp.asarray(ptr_seg, jnp.int32).reshape(1)
    kl3 = k_labels.reshape(NG, 1, C)
    seg3 = seg_labels.reshape(NBS, 1, C)
    queues, labels = _make_call()(
        pp, ps, queue_pcd, queue_seg, keys_pcd, keys_seg, kl3, seg3, seg3)
    new_ptr_pcd = ((jnp.asarray(ptr_pcd, jnp.int32) + BP) % K).astype(jnp.int32)
    new_ptr_seg = ((jnp.asarray(ptr_seg, jnp.int32) + BS) % K).astype(jnp.int32)
    return (queues, labels.reshape(K), new_ptr_pcd, new_ptr_seg)
